# baseline (speedup 1.0000x reference)
import functools

import jax
import jax.numpy as jnp
from jax import lax
from jax.experimental import pallas as pl
from jax.experimental.pallas import tpu as pltpu
from jax.experimental.pallas import tpu_sc as plsc

NUM_CHARS = 1000
EMB_DIM = 64
BATCH = 4096
SEQ = 20
N_ROWS = BATCH * SEQ


def _pmat_body(emb_ref, w_ref, b_ref, p_ref):
    p_ref[...] = lax.dot_general(
        emb_ref[...], w_ref[...],
        dimension_numbers=(((1,), (1,)), ((), ())),
        preferred_element_type=jnp.float32,
    ) + b_ref[...]


def _compute_p(emb_table, W, b):
    return pl.pallas_call(
        _pmat_body,
        out_shape=jax.ShapeDtypeStruct((NUM_CHARS, NUM_CHARS), jnp.float32),
    )(emb_table, W, b.reshape(1, NUM_CHARS))


_INFO = plsc.get_sparse_core_info()
_NC = _INFO.num_cores
_NS = _INFO.num_subcores
_NW = _NC * _NS
_RW = N_ROWS // _NW
_CHUNK = 64
_NCHUNK = _RW // _CHUNK


def _gather_body(p_hbm, idx_hbm, out_hbm, idx_v, rows_v, sem):
    wid = lax.axis_index("s") * _NC + lax.axis_index("c")
    base = wid * _RW
    pltpu.sync_copy(idx_hbm.at[pl.ds(base, _RW)], idx_v)

    def body(j, carry):
        off = pl.multiple_of(j * _CHUNK, 8)
        pltpu.async_copy(
            p_hbm.at[idx_v.at[pl.ds(off, _CHUNK)]], rows_v, sem).wait()
        pltpu.sync_copy(rows_v, out_hbm.at[pl.ds(base + off, _CHUNK)])
        return carry

    lax.fori_loop(0, _NCHUNK, body, 0)


@functools.partial(jax.jit, static_argnums=())
def _gather_rows(P, idx_flat):
    mesh = plsc.VectorSubcoreMesh(core_axis_name="c", subcore_axis_name="s")
    return pl.kernel(
        _gather_body,
        out_type=jax.ShapeDtypeStruct((N_ROWS, NUM_CHARS), jnp.float32),
        mesh=mesh,
        compiler_params=pltpu.CompilerParams(use_tc_tiling_on_sc=False),
        scratch_types=[
            pltpu.VMEM((_RW,), jnp.int32),
            pltpu.VMEM((_CHUNK, NUM_CHARS), jnp.float32),
            pltpu.SemaphoreType.DMA,
        ],
    )(P, idx_flat)


def kernel(x, emb_table, W, b):
    P = _compute_p(emb_table, W, b)
    out = _gather_rows(P, x.reshape(-1).astype(jnp.int32))
    return out.reshape(BATCH, SEQ, NUM_CHARS)


# speedup vs baseline: 1.0160x; 1.0160x over previous
import functools

import jax
import jax.numpy as jnp
from jax import lax
from jax.experimental import pallas as pl
from jax.experimental.pallas import tpu as pltpu
from jax.experimental.pallas import tpu_sc as plsc

NUM_CHARS = 1000
EMB_DIM = 64
BATCH = 4096
SEQ = 20
N_ROWS = BATCH * SEQ


def _pmat_body(emb_ref, w_ref, b_ref, p_ref):
    p_ref[...] = lax.dot_general(
        emb_ref[...], w_ref[...],
        dimension_numbers=(((1,), (1,)), ((), ())),
        preferred_element_type=jnp.float32,
    ) + b_ref[...]


def _compute_p(emb_table, W, b):
    return pl.pallas_call(
        _pmat_body,
        out_shape=jax.ShapeDtypeStruct((NUM_CHARS, NUM_CHARS), jnp.float32),
    )(emb_table, W, b.reshape(1, NUM_CHARS))


_INFO = plsc.get_sparse_core_info()
_NC = _INFO.num_cores
_NS = _INFO.num_subcores
_NW = _NC * _NS
_RW = N_ROWS // _NW
_CHUNK = 32
_NCHUNK = _RW // _CHUNK
_NB = 4
_NITER = _NCHUNK // _NB


def _gather_body(p_hbm, idx_hbm, out_hbm, idx_v, *scratch):
    bufs = scratch[:_NB]
    gsem = scratch[_NB:2 * _NB]
    ssem = scratch[2 * _NB:3 * _NB]
    wid = lax.axis_index("s") * _NC + lax.axis_index("c")
    base = wid * _RW
    pltpu.sync_copy(idx_hbm.at[pl.ds(base, _RW)], idx_v)

    def g_copy(b, c):
        off = pl.multiple_of(c * _CHUNK, 8)
        return pltpu.make_async_copy(
            p_hbm.at[idx_v.at[pl.ds(off, _CHUNK)]], bufs[b], gsem[b])

    def s_copy(b, c):
        off = pl.multiple_of(c * _CHUNK, 8)
        return pltpu.make_async_copy(
            bufs[b], out_hbm.at[pl.ds(base + off, _CHUNK)], ssem[b])

    for b in range(_NB):
        g_copy(b, b).start()

    def step(i, fire_next):
        for b in range(_NB):
            c = i * _NB + b
            g_copy(b, c).wait()
            s_copy(b, c).start()
            s_copy(b, c).wait()
            if fire_next:
                g_copy(b, c + _NB).start()
        return 0

    lax.fori_loop(0, _NITER - 1, lambda i, _: step(i, True), 0)
    step(_NITER - 1, False)


@functools.partial(jax.jit, static_argnums=())
def _gather_rows(P, idx_flat):
    mesh = plsc.VectorSubcoreMesh(core_axis_name="c", subcore_axis_name="s")
    return pl.kernel(
        _gather_body,
        out_type=jax.ShapeDtypeStruct((N_ROWS, NUM_CHARS), jnp.float32),
        mesh=mesh,
        compiler_params=pltpu.CompilerParams(use_tc_tiling_on_sc=False),
        scratch_types=[
            pltpu.VMEM((_RW,), jnp.int32),
        ] + [pltpu.VMEM((_CHUNK, NUM_CHARS), jnp.float32)] * _NB
          + [pltpu.SemaphoreType.DMA] * (2 * _NB),
    )(P, idx_flat)


def kernel(x, emb_table, W, b):
    P = _compute_p(emb_table, W, b)
    out = _gather_rows(P, x.reshape(-1).astype(jnp.int32))
    return out.reshape(BATCH, SEQ, NUM_CHARS)
